# baseline (speedup 1.0000x reference)
import functools

import jax
import jax.numpy as jnp
from jax import lax
from jax.experimental import pallas as pl
from jax.experimental.pallas import tpu as pltpu
from jax.experimental.pallas import tpu_sc as plsc

_VOCAB = 100000
_EMBED = 128
_BATCH = 1024
_CTX = 20
_HID = _EMBED // 2

_NC = 2
_NS = 16
_NW = _NC * _NS
_BAGS_PER_W = _BATCH // _NW
_TOK_PER_W = _BAGS_PER_W * _CTX
_CHUNK = 64
_NCHUNK = _TOK_PER_W // _CHUNK


def _pool_body(text_hbm, emb_hbm, out_hbm, idx_v, rows_v, pooled_v, sems):
    w = lax.axis_index("s") * _NC + lax.axis_index("c")
    pltpu.sync_copy(text_hbm.at[pl.ds(w * _TOK_PER_W, _TOK_PER_W)], idx_v)
    copies = [
        pltpu.async_copy(
            emb_hbm.at[idx_v.at[pl.ds(j * _CHUNK, _CHUNK)]],
            rows_v.at[pl.ds(j * _CHUNK, _CHUNK)],
            sems,
        )
        for j in range(_NCHUNK)
    ]
    for cp in copies:
        cp.wait()

    inv = 1.0 / _CTX

    def bag_body(b, carry):
        base = b * _CTX
        for v in range(_EMBED // 16):
            sl = pl.ds(v * 16, 16)
            acc = rows_v[base, sl]
            for r in range(1, _CTX):
                acc = acc + rows_v[base + r, sl]
            pooled_v[b, sl] = acc * inv
        return carry

    lax.fori_loop(0, _BAGS_PER_W, bag_body, 0)
    pltpu.sync_copy(pooled_v, out_hbm.at[pl.ds(w * _BAGS_PER_W, _BAGS_PER_W)])


_pool = functools.partial(
    pl.kernel,
    out_type=jax.ShapeDtypeStruct((_BATCH, _EMBED), jnp.float32),
    mesh=plsc.VectorSubcoreMesh(core_axis_name="c", subcore_axis_name="s"),
    scratch_types=[
        pltpu.VMEM((_TOK_PER_W,), jnp.int32),
        pltpu.VMEM((_TOK_PER_W, _EMBED), jnp.float32),
        pltpu.VMEM((_BAGS_PER_W, _EMBED), jnp.float32),
        pltpu.SemaphoreType.DMA,
    ],
)(_pool_body)


_VB = 4096
_GRID = (_VOCAB + _VB - 1) // _VB


def _mlp_body(pooled_ref, w1_ref, b1t_ref, wfct_ref, bfct_ref, outt_ref, ht_ref):
    @pl.when(pl.program_id(0) == 0)
    def _():
        ht = lax.dot_general(
            w1_ref[...], pooled_ref[...],
            (((1,), (1,)), ((), ())),
            preferred_element_type=jnp.float32,
        )
        ht_ref[...] = jnp.maximum(ht + b1t_ref[...], 0.0)

    outt_ref[...] = lax.dot_general(
        wfct_ref[...], ht_ref[...],
        (((0,), (0,)), ((), ())),
        preferred_element_type=jnp.float32,
    ) + jnp.transpose(bfct_ref[...])


_mlp = pl.pallas_call(
    _mlp_body,
    grid=(_GRID,),
    in_specs=[
        pl.BlockSpec((_BATCH, _EMBED), lambda i: (0, 0)),
        pl.BlockSpec((_HID, _EMBED), lambda i: (0, 0)),
        pl.BlockSpec((_HID, 1), lambda i: (0, 0)),
        pl.BlockSpec((_HID, _VB), lambda i: (0, i)),
        pl.BlockSpec((1, _VB), lambda i: (0, i)),
    ],
    out_specs=pl.BlockSpec((_VB, _BATCH), lambda i: (i, 0)),
    out_shape=jax.ShapeDtypeStruct((_VOCAB, _BATCH), jnp.float32),
    scratch_shapes=[pltpu.VMEM((_HID, _BATCH), jnp.float32)],
    compiler_params=pltpu.CompilerParams(dimension_semantics=("arbitrary",)),
)


def kernel(text, offsets, emb, W1, b1, Wfc, bfc):
    del offsets
    pooled = _pool(text, emb)
    logits_t = _mlp(
        pooled,
        W1,
        b1.reshape(_HID, 1),
        Wfc.T,
        bfc.reshape(1, _VOCAB),
    )
    return logits_t.T


# speedup vs baseline: 1.0015x; 1.0015x over previous
import functools

import jax
import jax.numpy as jnp
from jax import lax
from jax.experimental import pallas as pl
from jax.experimental.pallas import tpu as pltpu
from jax.experimental.pallas import tpu_sc as plsc

_VOCAB = 100000
_EMBED = 128
_BATCH = 1024
_CTX = 20
_HID = _EMBED // 2

_NC = 2
_NS = 16
_NW = _NC * _NS
_BAGS_PER_W = _BATCH // _NW
_TOK_PER_W = _BAGS_PER_W * _CTX
_CHUNK = 128
_NCHUNK = _TOK_PER_W // _CHUNK


def _pool_body(text_hbm, emb_hbm, out_hbm, idx_v, rows_v, pooled_v, sems):
    w = lax.axis_index("s") * _NC + lax.axis_index("c")
    pltpu.sync_copy(text_hbm.at[pl.ds(w * _TOK_PER_W, _TOK_PER_W)], idx_v)
    copies = [
        pltpu.async_copy(
            emb_hbm.at[idx_v.at[pl.ds(j * _CHUNK, _CHUNK)]],
            rows_v.at[pl.ds(j * _CHUNK, _CHUNK)],
            sems,
        )
        for j in range(_NCHUNK)
    ]
    for cp in copies:
        cp.wait()

    inv = 1.0 / _CTX

    def bag_body(b, carry):
        base = b * _CTX
        for v in range(_EMBED // 16):
            sl = pl.ds(v * 16, 16)
            acc = rows_v[base, sl]
            for r in range(1, _CTX):
                acc = acc + rows_v[base + r, sl]
            pooled_v[b, sl] = acc * inv
        return carry

    lax.fori_loop(0, _BAGS_PER_W, bag_body, 0)
    pltpu.sync_copy(pooled_v, out_hbm.at[pl.ds(w * _BAGS_PER_W, _BAGS_PER_W)])


_pool = functools.partial(
    pl.kernel,
    out_type=jax.ShapeDtypeStruct((_BATCH, _EMBED), jnp.float32),
    mesh=plsc.VectorSubcoreMesh(core_axis_name="c", subcore_axis_name="s"),
    scratch_types=[
        pltpu.VMEM((_TOK_PER_W,), jnp.int32),
        pltpu.VMEM((_TOK_PER_W, _EMBED), jnp.float32),
        pltpu.VMEM((_BAGS_PER_W, _EMBED), jnp.float32),
        pltpu.SemaphoreType.DMA,
    ],
)(_pool_body)


_VB = 4096
_GRID = (_VOCAB + _VB - 1) // _VB


def _mlp_body(pooled_ref, w1_ref, b1t_ref, wfct_ref, bfct_ref, outt_ref, ht_ref):
    @pl.when(pl.program_id(0) == 0)
    def _():
        ht = lax.dot_general(
            w1_ref[...], pooled_ref[...],
            (((1,), (1,)), ((), ())),
            preferred_element_type=jnp.float32,
        )
        ht_ref[...] = jnp.maximum(ht + b1t_ref[...], 0.0)

    outt_ref[...] = lax.dot_general(
        wfct_ref[...], ht_ref[...],
        (((0,), (0,)), ((), ())),
        preferred_element_type=jnp.float32,
    ) + jnp.transpose(bfct_ref[...])


_mlp = pl.pallas_call(
    _mlp_body,
    grid=(_GRID,),
    in_specs=[
        pl.BlockSpec((_BATCH, _EMBED), lambda i: (0, 0)),
        pl.BlockSpec((_HID, _EMBED), lambda i: (0, 0)),
        pl.BlockSpec((_HID, 1), lambda i: (0, 0)),
        pl.BlockSpec((_HID, _VB), lambda i: (0, i)),
        pl.BlockSpec((1, _VB), lambda i: (0, i)),
    ],
    out_specs=pl.BlockSpec((_VB, _BATCH), lambda i: (i, 0)),
    out_shape=jax.ShapeDtypeStruct((_VOCAB, _BATCH), jnp.float32),
    scratch_shapes=[pltpu.VMEM((_HID, _BATCH), jnp.float32)],
    compiler_params=pltpu.CompilerParams(dimension_semantics=("arbitrary",)),
)


def kernel(text, offsets, emb, W1, b1, Wfc, bfc):
    del offsets
    pooled = _pool(text, emb)
    logits_t = _mlp(
        pooled,
        W1,
        b1.reshape(_HID, 1),
        Wfc.T,
        bfc.reshape(1, _VOCAB),
    )
    return logits_t.T
